# baseline (speedup 1.0000x reference)
import functools

import jax
import jax.numpy as jnp
from jax import lax
from jax.experimental import pallas as pl
from jax.experimental.pallas import tpu as pltpu
from jax.experimental.pallas import tpu_sc as plsc

B, H, S, D, Q = 1, 8, 2048, 128, 16

NC, NS = 2, 16
NW = NC * NS
VWORDS = H * D * S
CH = VWORDS // NW
ROWS_W = CH // S


def _tc_body(pos_ref, *refs):
    ins = refs[0:8]
    outs = refs[8:12]
    pos = pos_ref[0]

    for l in range(4):
        ck, sk = ins[2 * l], ins[2 * l + 1]
        ko = outs[l]
        ko[...] = ck[...]
        for q in range(Q):
            ko[0, 0, pl.ds(pos + q, 1), :] = sk[0, 0, pl.ds(q, 1), :]


DW = 32


def _sc_body(pos_hbm, cv0, sv0, cv1, sv1, cv2, sv2, cv3, sv3,
             o0, o1, o2, o3, buf, svbuf, posv):
    cvs = (cv0, cv1, cv2, cv3)
    svs = (sv0, sv1, sv2, sv3)
    outs = (o0, o1, o2, o3)
    c = lax.axis_index("c")
    s = lax.axis_index("s")
    w = s * NC + c
    h = w // (D // DW)
    d0 = (w % (D // DW)) * DW

    pltpu.sync_copy(pos_hbm, posv)
    pos = posv[...][0]
    a = pl.multiple_of((pos // Q) * Q, Q)
    shift = pos - a
    li = lax.iota(jnp.int32, 16)
    idx = jnp.mod(li - shift, 16)

    for l in range(4):
        pltpu.sync_copy(cvs[l].at[0, h, pl.ds(d0, DW), :], buf)
        pltpu.sync_copy(svs[l].at[pl.ds(w * DW * Q, DW * Q)], svbuf)
        for r in range(DW):
            svrow = svbuf[pl.ds(r * Q, Q)]
            rot = svrow.at[idx].get(mode="promise_in_bounds")
            c0 = buf[r, pl.ds(a, Q)]
            c1 = buf[r, pl.ds(a + Q, Q)]
            buf[r, pl.ds(a, Q)] = jnp.where(li >= shift, rot, c0)
            buf[r, pl.ds(a + Q, Q)] = jnp.where(li < shift, rot, c1)
        pltpu.sync_copy(buf, outs[l].at[0, h, pl.ds(d0, DW), :])


def kernel(input_pos, kv_cache_k_0, kv_slice_k_0, kv_cache_v_0, kv_slice_v_0, kv_cache_k_1, kv_slice_k_1, kv_cache_v_1, kv_slice_v_1, kv_cache_k_2, kv_slice_k_2, kv_cache_v_2, kv_slice_v_2, kv_cache_k_3, kv_slice_k_3, kv_cache_v_3, kv_slice_v_3):
    pos32 = input_pos.astype(jnp.int32)
    ks = (
        kv_cache_k_0, kv_slice_k_0,
        kv_cache_k_1, kv_slice_k_1,
        kv_cache_k_2, kv_slice_k_2,
        kv_cache_k_3, kv_slice_k_3,
    )
    k_shape = jax.ShapeDtypeStruct((B, H, S, D), jnp.float32)
    k_cache_spec = pl.BlockSpec((B, 1, S, D), lambda h, p: (0, h, 0, 0))
    k_slice_spec = pl.BlockSpec((B, 1, Q, D), lambda h, p: (0, h, 0, 0))

    grid_spec = pltpu.PrefetchScalarGridSpec(
        num_scalar_prefetch=1,
        grid=(H,),
        in_specs=[k_cache_spec, k_slice_spec] * 4,
        out_specs=[k_cache_spec] * 4,
    )
    cvs = (kv_cache_v_0, kv_cache_v_1, kv_cache_v_2, kv_cache_v_3)
    svs = (kv_slice_v_0, kv_slice_v_1, kv_slice_v_2, kv_slice_v_3)
    v_shape = jax.ShapeDtypeStruct((B, H, D, S), jnp.float32)
    pos_vec = jnp.broadcast_to(pos32, (16,))

    sc_fn = functools.partial(
        pl.kernel,
        mesh=plsc.VectorSubcoreMesh(core_axis_name="c", subcore_axis_name="s"),
        out_type=(v_shape,) * 4,
        scratch_types=[
            pltpu.VMEM((DW, S), jnp.float32),
            pltpu.VMEM((DW * Q,), jnp.float32),
            pltpu.VMEM((16,), jnp.int32),
        ],
    )(_sc_body)
    sc_args = [pos_vec]
    for l in range(4):
        sc_args.append(cvs[l])
        sc_args.append(svs[l].reshape(H * D * Q))
    v_outs = sc_fn(*sc_args)

    k_outs = pl.pallas_call(
        _tc_body,
        grid_spec=grid_spec,
        out_shape=(k_shape,) * 4,
        compiler_params=pltpu.CompilerParams(
            dimension_semantics=("arbitrary",),
        ),
    )(pos32, *ks)

    return (
        k_outs[0], v_outs[0],
        k_outs[1], v_outs[1],
        k_outs[2], v_outs[2],
        k_outs[3], v_outs[3],
    )


# speedup vs baseline: 1.4015x; 1.4015x over previous
import jax
import jax.numpy as jnp
from jax.experimental import pallas as pl
from jax.experimental.pallas import tpu as pltpu

B, H, S, D, Q = 1, 8, 2048, 128, 16
C = 256
G = S // C


def _body(pos_ref, *refs):
    ins = refs[0:16]
    outs = refs[16:24]
    pos = pos_ref[0]
    i = pl.program_id(0)
    base = i * C

    for l in range(4):
        ck, sk, cv, sv = ins[4 * l], ins[4 * l + 1], ins[4 * l + 2], ins[4 * l + 3]
        ko, vo = outs[2 * l], outs[2 * l + 1]

        ko[...] = ck[...]
        r0 = pos - base
        for q in range(Q):
            rq = r0 + q

            @pl.when((rq >= 0) & (rq < C))
            def _(l=l, q=q, rq=rq, ko=ko, sk=sk):
                ko[0, :, pl.ds(jnp.clip(rq, 0, C - 1), 1), :] = sk[0, :, pl.ds(q, 1), :]

        vo[...] = cv[...]

        @pl.when((pos < base + C) & (pos + Q > base))
        def _(base=base, sv=sv, cv=cv, vo=vo):
            shift = jnp.mod(pos - base, C)
            padded = jnp.pad(sv[0][...], ((0, 0), (0, 0), (0, C - Q)))
            rolled = pltpu.roll(padded, shift, 2)
            lane_g = jax.lax.broadcasted_iota(jnp.int32, (1, 1, C), 2) + base
            mask = (lane_g >= pos) & (lane_g < pos + Q)
            vo[...] = jnp.where(mask[None], rolled[None], cv[...])


def kernel(input_pos, kv_cache_k_0, kv_slice_k_0, kv_cache_v_0, kv_slice_v_0, kv_cache_k_1, kv_slice_k_1, kv_cache_v_1, kv_slice_v_1, kv_cache_k_2, kv_slice_k_2, kv_cache_v_2, kv_slice_v_2, kv_cache_k_3, kv_slice_k_3, kv_cache_v_3, kv_slice_v_3):
    caches_and_slices = (
        kv_cache_k_0, kv_slice_k_0, kv_cache_v_0, kv_slice_v_0,
        kv_cache_k_1, kv_slice_k_1, kv_cache_v_1, kv_slice_v_1,
        kv_cache_k_2, kv_slice_k_2, kv_cache_v_2, kv_slice_v_2,
        kv_cache_k_3, kv_slice_k_3, kv_cache_v_3, kv_slice_v_3,
    )
    k_shape = jax.ShapeDtypeStruct((B, H, S, D), jnp.float32)
    v_shape = jax.ShapeDtypeStruct((B, H, D, S), jnp.float32)
    out_shape = (k_shape, v_shape) * 4

    k_cache_spec = pl.BlockSpec((B, H, C, D), lambda i, p: (0, 0, i, 0))
    k_slice_spec = pl.BlockSpec((B, H, Q, D), lambda i, p: (0, 0, 0, 0))
    v_cache_spec = pl.BlockSpec((B, H, D, C), lambda i, p: (0, 0, 0, i))
    v_slice_spec = pl.BlockSpec((B, H, D, Q), lambda i, p: (0, 0, 0, 0))

    grid_spec = pltpu.PrefetchScalarGridSpec(
        num_scalar_prefetch=1,
        grid=(G,),
        in_specs=[k_cache_spec, k_slice_spec, v_cache_spec, v_slice_spec] * 4,
        out_specs=[k_cache_spec, v_cache_spec] * 4,
    )

    outs = pl.pallas_call(
        _body,
        grid_spec=grid_spec,
        out_shape=out_shape,
        compiler_params=pltpu.CompilerParams(
            dimension_semantics=("arbitrary",),
        ),
    )(input_pos.astype(jnp.int32), *caches_and_slices)
    return tuple(outs)
